# baseline (speedup 1.0000x reference)
import functools

import jax
import jax.numpy as jnp
from jax import lax
from jax.experimental import pallas as pl
from jax.experimental.pallas import tpu as pltpu
from jax.experimental.pallas import tpu_sc as plsc

N = 10000
E = 320000
FH = 128
B = 64
C = 2

NC, NS = 2, 16
NW = NC * NS
N_PAD = 10240
ZR = N_PAD // NS
CH = 80
EPT = CH * 128
E_PAD = NW * EPT
NB = 2
GRP = 2
GC = CH // GRP


@functools.cache
def _sc_kernels():
    mesh = plsc.VectorSubcoreMesh(
        core_axis_name="c", subcore_axis_name="s",
        num_cores=NC, num_subcores=NS)

    @functools.partial(
        pl.kernel,
        out_type=jax.ShapeDtypeStruct((NC, N_PAD, FH), jnp.float32),
        mesh=mesh,
        scratch_types=[
            pltpu.VMEM((CH, 128), jnp.int32),
            pltpu.VMEM((128, FH), jnp.float32),
            pltpu.VMEM_SHARED((N_PAD, FH), jnp.float32),
            pltpu.SemaphoreType.DMA,
        ],
    )
    def sc_degree(dst_hbm, ones_hbm, zeros_hbm, out_hbm, dst_v, ones_v, acc,
                  sem):
        c = lax.axis_index("c")
        s = lax.axis_index("s")
        w = c * NS + s
        pltpu.sync_copy(zeros_hbm, acc.at[pl.ds(s * ZR, ZR)])
        pltpu.sync_copy(ones_hbm, ones_v)
        pltpu.sync_copy(dst_hbm.at[w], dst_v)
        plsc.subcore_barrier()

        def fire(j, carry):
            pltpu.async_copy(ones_v, acc.at[dst_v.at[j]], sem, add=True)
            return carry

        lax.fori_loop(0, CH, fire, 0)

        def drain(j, carry):
            pltpu.make_async_copy(ones_v, acc.at[dst_v.at[j]], sem).wait()
            return carry

        lax.fori_loop(0, CH, drain, 0)
        plsc.subcore_barrier()
        pltpu.sync_copy(acc.at[pl.ds(s * ZR, ZR)],
                        out_hbm.at[c, pl.ds(s * ZR, ZR)])

    @functools.partial(
        pl.kernel,
        out_type=jax.ShapeDtypeStruct((NC, N_PAD, FH), jnp.float32),
        mesh=mesh,
        scratch_types=[
            pltpu.VMEM((GC, 128), jnp.int32),
            pltpu.VMEM((GC, 128), jnp.int32),
        ]
        + [pltpu.VMEM((128, FH), jnp.float32)] * NB
        + [pltpu.VMEM_SHARED((N_PAD, FH), jnp.float32)]
        + [pltpu.SemaphoreType.DMA] * (2 * NB),
    )
    def sc_mp(p_hbm, src_hbm, dst_hbm, zeros_hbm, out_hbm,
              src_v, dst_v, *rest):
        rows = rest[:NB]
        acc = rest[NB]
        gsem = rest[NB + 1:NB + 1 + NB]
        ssem = rest[NB + 1 + NB:]
        c = lax.axis_index("c")
        s = lax.axis_index("s")
        w = c * NS + s
        pltpu.sync_copy(zeros_hbm, acc.at[pl.ds(s * ZR, ZR)])
        plsc.subcore_barrier()

        for grp in range(GRP):
            pltpu.sync_copy(src_hbm.at[w, pl.ds(grp * GC, GC)], src_v)
            pltpu.sync_copy(dst_hbm.at[w, pl.ds(grp * GC, GC)], dst_v)

            for b in range(NB):
                pltpu.async_copy(p_hbm.at[src_v.at[b]], rows[b], gsem[b])

            def outer(jj, carry):
                for b in range(NB):
                    i = jj * NB + b
                    bp = (b + 1) % NB
                    npre = i + 1

                    @pl.when(jnp.logical_and(npre >= NB, npre < GC))
                    def _():
                        pltpu.make_async_copy(
                            rows[bp], acc.at[dst_v.at[0]], ssem[bp]).wait()
                        pltpu.async_copy(p_hbm.at[src_v.at[npre]], rows[bp],
                                         gsem[bp])

                    pltpu.make_async_copy(p_hbm.at[src_v.at[i]], rows[b],
                                          gsem[b]).wait()
                    pltpu.async_copy(rows[b], acc.at[dst_v.at[i]], ssem[b],
                                     add=True)
                return carry

            lax.fori_loop(0, GC // NB, outer, 0)
            for b in range(NB):
                pltpu.make_async_copy(rows[b], acc.at[dst_v.at[0]],
                                      ssem[b]).wait()
        plsc.subcore_barrier()
        pltpu.sync_copy(acc.at[pl.ds(s * ZR, ZR)],
                        out_hbm.at[c, pl.ds(s * ZR, ZR)])

    return sc_degree, sc_mp


_BLK = 1024
_NBLK = N_PAD // _BLK

_row_spec = pl.BlockSpec((_BLK, FH), lambda i: (i, 0))
_deg_spec = pl.BlockSpec((NC, _BLK, FH), lambda i: (0, i, 0))
_w_spec = pl.BlockSpec((FH, FH), lambda i: (0, 0))
_b_spec = pl.BlockSpec((1, FH), lambda i: (0, 0))
_agg_spec = pl.BlockSpec((NC, _BLK, FH), lambda i: (0, i, 0))


def _afactor(deg_ref):
    d = deg_ref[0, :, 0:1] + deg_ref[1, :, 0:1] + 1.0
    return lax.rsqrt(d)


def _tc_prep_body(deg_ref, x_ref, w1_ref, b1_ref, w7_ref, b7_ref, p1_ref, p7_ref):
    a = _afactor(deg_ref)
    xb = x_ref[...]
    h1 = jnp.dot(xb, w1_ref[...], preferred_element_type=jnp.float32) + b1_ref[...]
    h7 = jnp.dot(xb, w7_ref[...], preferred_element_type=jnp.float32) + b7_ref[...]
    p1_ref[...] = a * h1
    p7_ref[...] = a * h7


_tc_prep = pl.pallas_call(
    _tc_prep_body,
    grid=(_NBLK,),
    in_specs=[_deg_spec, _row_spec, _w_spec, _b_spec, _w_spec, _b_spec],
    out_specs=[_row_spec, _row_spec],
    out_shape=[jax.ShapeDtypeStruct((N_PAD, FH), jnp.float32)] * 2,
)


def _tc_layer_body(deg_ref, agg_ref, p_ref, w_ref, b_ref, h_ref, pn_ref):
    a = _afactor(deg_ref)
    h = jnp.maximum(a * (agg_ref[0] + agg_ref[1] + p_ref[...]), 0.0)
    h_ref[...] = h
    pn_ref[...] = a * (jnp.dot(h, w_ref[...],               preferred_element_type=jnp.float32) + b_ref[...])


_tc_layer = pl.pallas_call(
    _tc_layer_body,
    grid=(_NBLK,),
    in_specs=[_deg_spec, _agg_spec, _row_spec, _w_spec, _b_spec],
    out_specs=[_row_spec, _row_spec],
    out_shape=[jax.ShapeDtypeStruct((N_PAD, FH), jnp.float32)] * 2,
)


def _tc_combine_body(deg_ref, agg_ref, p_ref, h_ref):
    a = _afactor(deg_ref)
    h_ref[...] = jnp.maximum(a * (agg_ref[0] + agg_ref[1] + p_ref[...]), 0.0)


_tc_combine = pl.pallas_call(
    _tc_combine_body,
    grid=(_NBLK,),
    in_specs=[_deg_spec, _agg_spec, _row_spec],
    out_specs=_row_spec,
    out_shape=jax.ShapeDtypeStruct((N_PAD, FH), jnp.float32),
)


def _tc_pool_body(h_ref, batch_ref, out_ref):
    b = pl.program_id(0)
    hv = h_ref[...]
    mask = batch_ref[...] == b
    mx = jnp.max(jnp.where(mask, hv, -jnp.inf), axis=0, keepdims=True)
    sm = jnp.sum(jnp.where(mask, hv, 0.0), axis=0, keepdims=True)
    cnt = jnp.sum(mask.astype(jnp.float32))
    mean = sm / jnp.maximum(cnt, 1.0)
    out_ref[...] = jnp.concatenate([mx, mean], axis=1).reshape(1, 1, 2 * FH)


_tc_pool = pl.pallas_call(
    _tc_pool_body,
    grid=(B,),
    in_specs=[
        pl.BlockSpec((N_PAD, FH), lambda b: (0, 0)),
        pl.BlockSpec((N_PAD, 1), lambda b: (0, 0)),
    ],
    out_specs=pl.BlockSpec((1, 1, 2 * FH), lambda b: (b, 0, 0)),
    out_shape=jax.ShapeDtypeStruct((B, 1, 2 * FH), jnp.float32),
)


def _tc_head_body(x1, x2, x3, pa, y, l1, c1, l2, c2, l3, c3,
                  loss_ref, pred_ref, probs_ref, xaug_ref):
    r = (jnp.maximum(x1[...], 0.0) + jnp.maximum(x2[...], 0.0)
         + jnp.maximum(x3[...], 0.0))
    z = jnp.maximum(jnp.dot(r, l1[...],            preferred_element_type=jnp.float32) + c1[...], 0.0)
    z = jnp.maximum(jnp.dot(z, l2[...],            preferred_element_type=jnp.float32) + c2[...], 0.0)
    lg = jnp.dot(z, l3[...], preferred_element_type=jnp.float32) + c3[...]
    lo = lg[:, 0:1]
    hi = lg[:, 1:2]
    m = jnp.maximum(lo, hi)
    lse = m + jnp.log(jnp.exp(lo - m) + jnp.exp(hi - m))
    lp0 = lo - lse
    lp1 = hi - lse
    lpy = jnp.where(y[...] == 1, lp1, lp0)
    loss_ref[...] = jnp.reshape(-jnp.mean(lpy), (1, 1))
    pred_ref[...] = (hi > lo).astype(jnp.int32)
    probs_ref[...] = jnp.concatenate([jnp.exp(lp0), jnp.exp(lp1)], axis=1)
    pav = pa[...]
    for k in range(10):
        xaug_ref[:, 2 * FH * k:2 * FH * (k + 1)] = pav


_tc_head = pl.pallas_call(
    _tc_head_body,
    out_shape=[
        jax.ShapeDtypeStruct((1, 1), jnp.float32),
        jax.ShapeDtypeStruct((B, 1), jnp.int32),
        jax.ShapeDtypeStruct((B, C), jnp.float32),
        jax.ShapeDtypeStruct((B, 20 * FH), jnp.float32),
    ],
)


def kernel(x, edge_index, batch, y, W1, b1, W2, b2, W3, b3, W7, b7, W8, b8,
           L1, c1, L2, c2, L3, c3):
    src = edge_index[0]
    dst = edge_index[1]
    pad = jnp.full((E_PAD - E,), N, dtype=jnp.int32)
    src3 = jnp.concatenate([src, pad]).reshape(NW, CH, 128)
    dst3 = jnp.concatenate([dst, pad]).reshape(NW, CH, 128)
    x_pad = jnp.pad(x, ((0, N_PAD - N), (0, 0)))
    batch_col = jnp.pad(batch, (0, N_PAD - N), constant_values=B).reshape(N_PAD, 1)
    zeros128 = jnp.zeros((ZR, FH), jnp.float32)
    ones128 = jnp.ones((128, FH), jnp.float32)
    row = lambda v: v.reshape(1, -1)
    _sc_degree, _sc_mp = _sc_kernels()

    deg2 = _sc_degree(dst3, ones128, zeros128)
    p1, p7 = _tc_prep(deg2, x_pad, W1, row(b1), W7, row(b7))

    agg1 = _sc_mp(p1, src3, dst3, zeros128)
    agg7 = _sc_mp(p7, src3, dst3, zeros128)

    h1, p2 = _tc_layer(deg2, agg1, p1, W2, row(b2))
    x1 = _tc_pool(h1, batch_col)
    agg2 = _sc_mp(p2, src3, dst3, zeros128)

    h2, p3 = _tc_layer(deg2, agg2, p2, W3, row(b3))
    x2 = _tc_pool(h2, batch_col)
    agg3 = _sc_mp(p3, src3, dst3, zeros128)

    h3 = _tc_combine(deg2, agg3, p3)
    x3 = _tc_pool(h3, batch_col)

    ha1, p8 = _tc_layer(deg2, agg7, p7, W8, row(b8))
    agg8 = _sc_mp(p8, src3, dst3, zeros128)
    ha2 = _tc_combine(deg2, agg8, p8)
    pa = _tc_pool(ha2, batch_col)

    sq = lambda v: v.reshape(B, 2 * FH)
    loss, pred, probs, xaug = _tc_head(
        sq(x1), sq(x2), sq(x3), sq(pa), y.reshape(B, 1),
        L1, row(c1), L2, row(c2), L3, row(c3))
    return (loss.reshape(()), pred.reshape(B), probs, xaug)


# speedup vs baseline: 1.0012x; 1.0012x over previous
import functools

import jax
import jax.numpy as jnp
from jax import lax
from jax.experimental import pallas as pl
from jax.experimental.pallas import tpu as pltpu
from jax.experimental.pallas import tpu_sc as plsc

N = 10000
E = 320000
FH = 128
B = 64
C = 2

NC, NS = 2, 16
NW = NC * NS
N_PAD = 10240
ZR = N_PAD // NS
CH = 80
EPT = CH * 128
E_PAD = NW * EPT
NB = 2
GRP = 2
GC = CH // GRP


@functools.cache
def _sc_kernels():
    mesh = plsc.VectorSubcoreMesh(
        core_axis_name="c", subcore_axis_name="s",
        num_cores=NC, num_subcores=NS)

    @functools.partial(
        pl.kernel,
        out_type=jax.ShapeDtypeStruct((NC, N_PAD, FH), jnp.float32),
        mesh=mesh,
        scratch_types=[
            pltpu.VMEM((CH, 128), jnp.int32),
            pltpu.VMEM((128, FH), jnp.float32),
            pltpu.VMEM_SHARED((N_PAD, FH), jnp.float32),
            pltpu.SemaphoreType.DMA,
        ],
    )
    def sc_degree(dst_hbm, ones_hbm, zeros_hbm, out_hbm, dst_v, ones_v, acc,
                  sem):
        c = lax.axis_index("c")
        s = lax.axis_index("s")
        w = c * NS + s
        pltpu.sync_copy(zeros_hbm, acc.at[pl.ds(s * ZR, ZR)])
        pltpu.sync_copy(ones_hbm, ones_v)
        pltpu.sync_copy(dst_hbm.at[w], dst_v)
        plsc.subcore_barrier()

        def fire(j, carry):
            pltpu.sync_copy(ones_v, acc.at[dst_v.at[j]], add=True)
            return carry

        lax.fori_loop(0, CH, fire, 0)
        plsc.subcore_barrier()
        pltpu.sync_copy(acc.at[pl.ds(s * ZR, ZR)],
                        out_hbm.at[c, pl.ds(s * ZR, ZR)])

    @functools.partial(
        pl.kernel,
        out_type=jax.ShapeDtypeStruct((NC, N_PAD, FH), jnp.float32),
        mesh=mesh,
        scratch_types=[
            pltpu.VMEM((GC, 128), jnp.int32),
            pltpu.VMEM((GC, 128), jnp.int32),
        ]
        + [pltpu.VMEM((128, FH), jnp.float32)] * NB
        + [pltpu.VMEM_SHARED((N_PAD, FH), jnp.float32)]
        + [pltpu.SemaphoreType.DMA] * (2 * NB),
    )
    def sc_mp(p_hbm, src_hbm, dst_hbm, zeros_hbm, out_hbm,
              src_v, dst_v, *rest):
        rows = rest[:NB]
        acc = rest[NB]
        gsem = rest[NB + 1:NB + 1 + NB]
        ssem = rest[NB + 1 + NB:]
        c = lax.axis_index("c")
        s = lax.axis_index("s")
        w = c * NS + s
        pltpu.sync_copy(zeros_hbm, acc.at[pl.ds(s * ZR, ZR)])
        plsc.subcore_barrier()

        for grp in range(GRP):
            pltpu.sync_copy(src_hbm.at[w, pl.ds(grp * GC, GC)], src_v)
            pltpu.sync_copy(dst_hbm.at[w, pl.ds(grp * GC, GC)], dst_v)

            for b in range(NB):
                pltpu.async_copy(p_hbm.at[src_v.at[b]], rows[b], gsem[b])

            def outer(jj, carry):
                for b in range(NB):
                    i = jj * NB + b
                    bp = (b + 1) % NB
                    npre = i + 1

                    @pl.when(jnp.logical_and(npre >= NB, npre < GC))
                    def _():
                        pltpu.make_async_copy(
                            rows[bp], acc.at[dst_v.at[0]], ssem[bp]).wait()
                        pltpu.async_copy(p_hbm.at[src_v.at[npre]], rows[bp],
                                         gsem[bp])

                    pltpu.make_async_copy(p_hbm.at[src_v.at[i]], rows[b],
                                          gsem[b]).wait()
                    pltpu.async_copy(rows[b], acc.at[dst_v.at[i]], ssem[b],
                                     add=True)
                return carry

            lax.fori_loop(0, GC // NB, outer, 0)
            for b in range(NB):
                pltpu.make_async_copy(rows[b], acc.at[dst_v.at[0]],
                                      ssem[b]).wait()
        plsc.subcore_barrier()
        pltpu.sync_copy(acc.at[pl.ds(s * ZR, ZR)],
                        out_hbm.at[c, pl.ds(s * ZR, ZR)])

    return sc_degree, sc_mp


_BLK = 1024
_NBLK = N_PAD // _BLK

_row_spec = pl.BlockSpec((_BLK, FH), lambda i: (i, 0))
_deg_spec = pl.BlockSpec((NC, _BLK, FH), lambda i: (0, i, 0))
_w_spec = pl.BlockSpec((FH, FH), lambda i: (0, 0))
_b_spec = pl.BlockSpec((1, FH), lambda i: (0, 0))
_agg_spec = pl.BlockSpec((NC, _BLK, FH), lambda i: (0, i, 0))


def _afactor(deg_ref):
    d = deg_ref[0, :, 0:1] + deg_ref[1, :, 0:1] + 1.0
    return lax.rsqrt(d)


def _tc_prep_body(deg_ref, x_ref, w1_ref, b1_ref, w7_ref, b7_ref, p1_ref, p7_ref):
    a = _afactor(deg_ref)
    xb = x_ref[...]
    h1 = jnp.dot(xb, w1_ref[...], preferred_element_type=jnp.float32) + b1_ref[...]
    h7 = jnp.dot(xb, w7_ref[...], preferred_element_type=jnp.float32) + b7_ref[...]
    p1_ref[...] = a * h1
    p7_ref[...] = a * h7


_tc_prep = pl.pallas_call(
    _tc_prep_body,
    grid=(_NBLK,),
    in_specs=[_deg_spec, _row_spec, _w_spec, _b_spec, _w_spec, _b_spec],
    out_specs=[_row_spec, _row_spec],
    out_shape=[jax.ShapeDtypeStruct((N_PAD, FH), jnp.float32)] * 2,
)


def _tc_layer_body(deg_ref, agg_ref, p_ref, w_ref, b_ref, h_ref, pn_ref):
    a = _afactor(deg_ref)
    h = jnp.maximum(a * (agg_ref[0] + agg_ref[1] + p_ref[...]), 0.0)
    h_ref[...] = h
    pn_ref[...] = a * (jnp.dot(h, w_ref[...],               preferred_element_type=jnp.float32) + b_ref[...])


_tc_layer = pl.pallas_call(
    _tc_layer_body,
    grid=(_NBLK,),
    in_specs=[_deg_spec, _agg_spec, _row_spec, _w_spec, _b_spec],
    out_specs=[_row_spec, _row_spec],
    out_shape=[jax.ShapeDtypeStruct((N_PAD, FH), jnp.float32)] * 2,
)


def _tc_combine_body(deg_ref, agg_ref, p_ref, h_ref):
    a = _afactor(deg_ref)
    h_ref[...] = jnp.maximum(a * (agg_ref[0] + agg_ref[1] + p_ref[...]), 0.0)


_tc_combine = pl.pallas_call(
    _tc_combine_body,
    grid=(_NBLK,),
    in_specs=[_deg_spec, _agg_spec, _row_spec],
    out_specs=_row_spec,
    out_shape=jax.ShapeDtypeStruct((N_PAD, FH), jnp.float32),
)


def _tc_pool_body(h_ref, batch_ref, out_ref):
    b = pl.program_id(0)
    hv = h_ref[...]
    mask = batch_ref[...] == b
    mx = jnp.max(jnp.where(mask, hv, -jnp.inf), axis=0, keepdims=True)
    sm = jnp.sum(jnp.where(mask, hv, 0.0), axis=0, keepdims=True)
    cnt = jnp.sum(mask.astype(jnp.float32))
    mean = sm / jnp.maximum(cnt, 1.0)
    out_ref[...] = jnp.concatenate([mx, mean], axis=1).reshape(1, 1, 2 * FH)


_tc_pool = pl.pallas_call(
    _tc_pool_body,
    grid=(B,),
    in_specs=[
        pl.BlockSpec((N_PAD, FH), lambda b: (0, 0)),
        pl.BlockSpec((N_PAD, 1), lambda b: (0, 0)),
    ],
    out_specs=pl.BlockSpec((1, 1, 2 * FH), lambda b: (b, 0, 0)),
    out_shape=jax.ShapeDtypeStruct((B, 1, 2 * FH), jnp.float32),
)


def _tc_head_body(x1, x2, x3, pa, y, l1, c1, l2, c2, l3, c3,
                  loss_ref, pred_ref, probs_ref, xaug_ref):
    r = (jnp.maximum(x1[...], 0.0) + jnp.maximum(x2[...], 0.0)
         + jnp.maximum(x3[...], 0.0))
    z = jnp.maximum(jnp.dot(r, l1[...],            preferred_element_type=jnp.float32) + c1[...], 0.0)
    z = jnp.maximum(jnp.dot(z, l2[...],            preferred_element_type=jnp.float32) + c2[...], 0.0)
    lg = jnp.dot(z, l3[...], preferred_element_type=jnp.float32) + c3[...]
    lo = lg[:, 0:1]
    hi = lg[:, 1:2]
    m = jnp.maximum(lo, hi)
    lse = m + jnp.log(jnp.exp(lo - m) + jnp.exp(hi - m))
    lp0 = lo - lse
    lp1 = hi - lse
    lpy = jnp.where(y[...] == 1, lp1, lp0)
    loss_ref[...] = jnp.reshape(-jnp.mean(lpy), (1, 1))
    pred_ref[...] = (hi > lo).astype(jnp.int32)
    probs_ref[...] = jnp.concatenate([jnp.exp(lp0), jnp.exp(lp1)], axis=1)
    pav = pa[...]
    for k in range(10):
        xaug_ref[:, 2 * FH * k:2 * FH * (k + 1)] = pav


_tc_head = pl.pallas_call(
    _tc_head_body,
    out_shape=[
        jax.ShapeDtypeStruct((1, 1), jnp.float32),
        jax.ShapeDtypeStruct((B, 1), jnp.int32),
        jax.ShapeDtypeStruct((B, C), jnp.float32),
        jax.ShapeDtypeStruct((B, 20 * FH), jnp.float32),
    ],
)


def kernel(x, edge_index, batch, y, W1, b1, W2, b2, W3, b3, W7, b7, W8, b8,
           L1, c1, L2, c2, L3, c3):
    src = edge_index[0]
    dst = edge_index[1]
    pad = jnp.full((E_PAD - E,), N, dtype=jnp.int32)
    src3 = jnp.concatenate([src, pad]).reshape(NW, CH, 128)
    dst3 = jnp.concatenate([dst, pad]).reshape(NW, CH, 128)
    x_pad = jnp.pad(x, ((0, N_PAD - N), (0, 0)))
    batch_col = jnp.pad(batch, (0, N_PAD - N), constant_values=B).reshape(N_PAD, 1)
    zeros128 = jnp.zeros((ZR, FH), jnp.float32)
    ones128 = jnp.ones((128, FH), jnp.float32)
    row = lambda v: v.reshape(1, -1)
    _sc_degree, _sc_mp = _sc_kernels()

    deg2 = _sc_degree(dst3, ones128, zeros128)
    p1, p7 = _tc_prep(deg2, x_pad, W1, row(b1), W7, row(b7))

    agg1 = _sc_mp(p1, src3, dst3, zeros128)
    agg7 = _sc_mp(p7, src3, dst3, zeros128)

    h1, p2 = _tc_layer(deg2, agg1, p1, W2, row(b2))
    x1 = _tc_pool(h1, batch_col)
    agg2 = _sc_mp(p2, src3, dst3, zeros128)

    h2, p3 = _tc_layer(deg2, agg2, p2, W3, row(b3))
    x2 = _tc_pool(h2, batch_col)
    agg3 = _sc_mp(p3, src3, dst3, zeros128)

    h3 = _tc_combine(deg2, agg3, p3)
    x3 = _tc_pool(h3, batch_col)

    ha1, p8 = _tc_layer(deg2, agg7, p7, W8, row(b8))
    agg8 = _sc_mp(p8, src3, dst3, zeros128)
    ha2 = _tc_combine(deg2, agg8, p8)
    pa = _tc_pool(ha2, batch_col)

    sq = lambda v: v.reshape(B, 2 * FH)
    loss, pred, probs, xaug = _tc_head(
        sq(x1), sq(x2), sq(x3), sq(pa), y.reshape(B, 1),
        L1, row(c1), L2, row(c2), L3, row(c3))
    return (loss.reshape(()), pred.reshape(B), probs, xaug)


# speedup vs baseline: 2.5326x; 2.5297x over previous
import functools

import jax
import jax.numpy as jnp
from jax import lax
from jax.experimental import pallas as pl
from jax.experimental.pallas import tpu as pltpu
from jax.experimental.pallas import tpu_sc as plsc

N = 10000
E = 320000
FH = 128
B = 64
C = 2

NC, NS = 2, 16
NW = NC * NS
N_PAD = 10240
ZR = N_PAD // NS
CH = 80
EPT = CH * 128
E_PAD = NW * EPT
NB = 2
GRP = 2
GC = CH // GRP


@functools.cache
def _sc_kernels():
    mesh = plsc.VectorSubcoreMesh(
        core_axis_name="c", subcore_axis_name="s",
        num_cores=NC, num_subcores=NS)

    @functools.partial(
        pl.kernel,
        out_type=jax.ShapeDtypeStruct((NC, N_PAD, FH), jnp.float32),
        mesh=mesh,
        scratch_types=[
            pltpu.VMEM((CH, 128), jnp.int32),
            pltpu.VMEM((128, FH), jnp.float32),
            pltpu.VMEM_SHARED((N_PAD, FH), jnp.float32),
            pltpu.SemaphoreType.DMA,
        ],
    )
    def sc_degree(dst_hbm, ones_hbm, zeros_hbm, out_hbm, dst_v, ones_v, acc,
                  sem):
        c = lax.axis_index("c")
        s = lax.axis_index("s")
        w = c * NS + s
        pltpu.sync_copy(zeros_hbm, acc.at[pl.ds(s * ZR, ZR)])
        pltpu.sync_copy(ones_hbm, ones_v)
        pltpu.sync_copy(dst_hbm.at[w], dst_v)
        plsc.subcore_barrier()

        def fire(j, carry):
            pltpu.sync_copy(ones_v, acc.at[dst_v.at[j]], add=True)
            return carry

        lax.fori_loop(0, CH, fire, 0)
        plsc.subcore_barrier()
        pltpu.sync_copy(acc.at[pl.ds(s * ZR, ZR)],
                        out_hbm.at[c, pl.ds(s * ZR, ZR)])

    @functools.partial(
        pl.kernel,
        out_type=jax.ShapeDtypeStruct((NC, N_PAD, FH), jnp.float32),
        mesh=mesh,
        scratch_types=[
            pltpu.VMEM((GC, 128), jnp.int32),
            pltpu.VMEM((GC, 128), jnp.int32),
        ]
        + [pltpu.VMEM((128, FH), jnp.float32)] * NB
        + [pltpu.VMEM_SHARED((N_PAD, FH), jnp.float32)]
        + [pltpu.SemaphoreType.DMA] * (2 * NB),
    )
    def sc_mp(p_hbm, src_hbm, dst_hbm, zeros_hbm, out_hbm,
              src_v, dst_v, *rest):
        rows = rest[:NB]
        acc = rest[NB]
        gsem = rest[NB + 1:NB + 1 + NB]
        ssem = rest[NB + 1 + NB:]
        c = lax.axis_index("c")
        s = lax.axis_index("s")
        w = c * NS + s
        pltpu.sync_copy(zeros_hbm, acc.at[pl.ds(s * ZR, ZR)])
        plsc.subcore_barrier()

        for grp in range(GRP):
            pltpu.sync_copy(src_hbm.at[w, pl.ds(grp * GC, GC)], src_v)
            pltpu.sync_copy(dst_hbm.at[w, pl.ds(grp * GC, GC)], dst_v)

            for b in range(NB):
                pltpu.async_copy(p_hbm.at[src_v.at[b]], rows[b], gsem[b])

            def outer(jj, carry):
                for b in range(NB):
                    i = jj * NB + b
                    bp = (b + 1) % NB
                    npre = i + 1

                    @pl.when(jnp.logical_and(npre >= NB, npre < GC))
                    def _():
                        pltpu.make_async_copy(
                            rows[bp], acc.at[dst_v.at[0]], ssem[bp]).wait()
                        pltpu.async_copy(p_hbm.at[src_v.at[npre]], rows[bp],
                                         gsem[bp])

                    pltpu.make_async_copy(p_hbm.at[src_v.at[i]], rows[b],
                                          gsem[b]).wait()
                    pltpu.async_copy(rows[b], acc.at[dst_v.at[i]], ssem[b],
                                     add=True)
                return carry

            lax.fori_loop(0, GC // NB, outer, 0)
            for b in range(NB):
                pltpu.make_async_copy(rows[b], acc.at[dst_v.at[0]],
                                      ssem[b]).wait()
        plsc.subcore_barrier()
        pltpu.sync_copy(acc.at[pl.ds(s * ZR, ZR)],
                        out_hbm.at[c, pl.ds(s * ZR, ZR)])

    return sc_degree, sc_mp


_BLK = 1024
_NBLK = N_PAD // _BLK

_row_spec = pl.BlockSpec((_BLK, FH), lambda i: (i, 0))
_deg_spec = pl.BlockSpec((NC, _BLK, FH), lambda i: (0, i, 0))
_w_spec = pl.BlockSpec((FH, FH), lambda i: (0, 0))
_b_spec = pl.BlockSpec((1, FH), lambda i: (0, 0))
_agg_spec = pl.BlockSpec((NC, _BLK, FH), lambda i: (0, i, 0))


def _afactor(deg_ref):
    d = deg_ref[0, :, 0:1] + deg_ref[1, :, 0:1] + 1.0
    return lax.rsqrt(d)


def _tc_prep_body(deg_ref, x_ref, w1_ref, b1_ref, w7_ref, b7_ref, p1_ref, p7_ref):
    a = _afactor(deg_ref)
    xb = x_ref[...]
    h1 = jnp.dot(xb, w1_ref[...], preferred_element_type=jnp.float32) + b1_ref[...]
    h7 = jnp.dot(xb, w7_ref[...], preferred_element_type=jnp.float32) + b7_ref[...]
    p1_ref[...] = a * h1
    p7_ref[...] = a * h7


_tc_prep = pl.pallas_call(
    _tc_prep_body,
    grid=(_NBLK,),
    in_specs=[_deg_spec, _row_spec, _w_spec, _b_spec, _w_spec, _b_spec],
    out_specs=[_row_spec, _row_spec],
    out_shape=[jax.ShapeDtypeStruct((N_PAD, FH), jnp.float32)] * 2,
)


def _tc_layer_body(deg_ref, agg_ref, p_ref, w_ref, b_ref, h_ref, pn_ref):
    a = _afactor(deg_ref)
    h = jnp.maximum(a * (agg_ref[0] + agg_ref[1] + p_ref[...]), 0.0)
    h_ref[...] = h
    pn_ref[...] = a * (jnp.dot(h, w_ref[...],               preferred_element_type=jnp.float32) + b_ref[...])


_tc_layer = pl.pallas_call(
    _tc_layer_body,
    grid=(_NBLK,),
    in_specs=[_deg_spec, _agg_spec, _row_spec, _w_spec, _b_spec],
    out_specs=[_row_spec, _row_spec],
    out_shape=[jax.ShapeDtypeStruct((N_PAD, FH), jnp.float32)] * 2,
)


def _tc_combine_body(deg_ref, agg_ref, p_ref, h_ref):
    a = _afactor(deg_ref)
    h_ref[...] = jnp.maximum(a * (agg_ref[0] + agg_ref[1] + p_ref[...]), 0.0)


_tc_combine = pl.pallas_call(
    _tc_combine_body,
    grid=(_NBLK,),
    in_specs=[_deg_spec, _agg_spec, _row_spec],
    out_specs=_row_spec,
    out_shape=jax.ShapeDtypeStruct((N_PAD, FH), jnp.float32),
)


def _tc_pool_body(h_ref, batch_ref, out_ref):
    b = pl.program_id(0)
    hv = h_ref[...]
    mask = batch_ref[...] == b
    mx = jnp.max(jnp.where(mask, hv, -jnp.inf), axis=0, keepdims=True)
    sm = jnp.sum(jnp.where(mask, hv, 0.0), axis=0, keepdims=True)
    cnt = jnp.sum(mask.astype(jnp.float32))
    mean = sm / jnp.maximum(cnt, 1.0)
    out_ref[...] = jnp.concatenate([mx, mean], axis=1).reshape(1, 1, 2 * FH)


_tc_pool = pl.pallas_call(
    _tc_pool_body,
    grid=(B,),
    in_specs=[
        pl.BlockSpec((N_PAD, FH), lambda b: (0, 0)),
        pl.BlockSpec((N_PAD, 1), lambda b: (0, 0)),
    ],
    out_specs=pl.BlockSpec((1, 1, 2 * FH), lambda b: (b, 0, 0)),
    out_shape=jax.ShapeDtypeStruct((B, 1, 2 * FH), jnp.float32),
)


def _tc_head_body(x1, x2, x3, pa, y, l1, c1, l2, c2, l3, c3,
                  loss_ref, pred_ref, probs_ref, xaug_ref):
    r = (jnp.maximum(x1[...], 0.0) + jnp.maximum(x2[...], 0.0)
         + jnp.maximum(x3[...], 0.0))
    z = jnp.maximum(jnp.dot(r, l1[...],            preferred_element_type=jnp.float32) + c1[...], 0.0)
    z = jnp.maximum(jnp.dot(z, l2[...],            preferred_element_type=jnp.float32) + c2[...], 0.0)
    lg = jnp.dot(z, l3[...], preferred_element_type=jnp.float32) + c3[...]
    lo = lg[:, 0:1]
    hi = lg[:, 1:2]
    m = jnp.maximum(lo, hi)
    lse = m + jnp.log(jnp.exp(lo - m) + jnp.exp(hi - m))
    lp0 = lo - lse
    lp1 = hi - lse
    lpy = jnp.where(y[...] == 1, lp1, lp0)
    loss_ref[...] = jnp.reshape(-jnp.mean(lpy), (1, 1))
    pred_ref[...] = (hi > lo).astype(jnp.int32)
    probs_ref[...] = jnp.concatenate([jnp.exp(lp0), jnp.exp(lp1)], axis=1)
    pav = pa[...]
    for k in range(10):
        xaug_ref[:, 2 * FH * k:2 * FH * (k + 1)] = pav


_tc_head = pl.pallas_call(
    _tc_head_body,
    out_shape=[
        jax.ShapeDtypeStruct((1, 1), jnp.float32),
        jax.ShapeDtypeStruct((B, 1), jnp.int32),
        jax.ShapeDtypeStruct((B, C), jnp.float32),
        jax.ShapeDtypeStruct((B, 20 * FH), jnp.float32),
    ],
)


def kernel(x, edge_index, batch, y, W1, b1, W2, b2, W3, b3, W7, b7, W8, b8,
           L1, c1, L2, c2, L3, c3):
    src = edge_index[0]
    dst = edge_index[1]
    pad = (jnp.arange(E_PAD - E, dtype=jnp.int32) % (N_PAD - N)) + N
    src3 = jnp.concatenate([src, pad]).reshape(NW, CH, 128)
    dst3 = jnp.concatenate([dst, pad]).reshape(NW, CH, 128)
    x_pad = jnp.pad(x, ((0, N_PAD - N), (0, 0)))
    batch_col = jnp.pad(batch, (0, N_PAD - N), constant_values=B).reshape(N_PAD, 1)
    zeros128 = jnp.zeros((ZR, FH), jnp.float32)
    ones128 = jnp.ones((128, FH), jnp.float32)
    row = lambda v: v.reshape(1, -1)
    _sc_degree, _sc_mp = _sc_kernels()

    deg2 = _sc_degree(dst3, ones128, zeros128)
    p1, p7 = _tc_prep(deg2, x_pad, W1, row(b1), W7, row(b7))

    agg1 = _sc_mp(p1, src3, dst3, zeros128)
    agg7 = _sc_mp(p7, src3, dst3, zeros128)

    h1, p2 = _tc_layer(deg2, agg1, p1, W2, row(b2))
    x1 = _tc_pool(h1, batch_col)
    agg2 = _sc_mp(p2, src3, dst3, zeros128)

    h2, p3 = _tc_layer(deg2, agg2, p2, W3, row(b3))
    x2 = _tc_pool(h2, batch_col)
    agg3 = _sc_mp(p3, src3, dst3, zeros128)

    h3 = _tc_combine(deg2, agg3, p3)
    x3 = _tc_pool(h3, batch_col)

    ha1, p8 = _tc_layer(deg2, agg7, p7, W8, row(b8))
    agg8 = _sc_mp(p8, src3, dst3, zeros128)
    ha2 = _tc_combine(deg2, agg8, p8)
    pa = _tc_pool(ha2, batch_col)

    sq = lambda v: v.reshape(B, 2 * FH)
    loss, pred, probs, xaug = _tc_head(
        sq(x1), sq(x2), sq(x3), sq(pa), y.reshape(B, 1),
        L1, row(c1), L2, row(c2), L3, row(c3))
    return (loss.reshape(()), pred.reshape(B), probs, xaug)


# speedup vs baseline: 2.5367x; 1.0016x over previous
import functools

import jax
import jax.numpy as jnp
from jax import lax
from jax.experimental import pallas as pl
from jax.experimental.pallas import tpu as pltpu
from jax.experimental.pallas import tpu_sc as plsc

N = 10000
E = 320000
FH = 128
B = 64
C = 2

NC, NS = 2, 16
NW = NC * NS
N_PAD = 10240
ZR = N_PAD // NS
CH = 80
EPT = CH * 128
E_PAD = NW * EPT
NB = 2
GRP = 2
GC = CH // GRP


@functools.cache
def _sc_kernels():
    mesh = plsc.VectorSubcoreMesh(
        core_axis_name="c", subcore_axis_name="s",
        num_cores=NC, num_subcores=NS)

    @functools.partial(
        pl.kernel,
        out_type=jax.ShapeDtypeStruct((NC, N_PAD, FH), jnp.float32),
        mesh=mesh,
        scratch_types=[
            pltpu.VMEM((CH, 128), jnp.int32),
            pltpu.VMEM((128, FH), jnp.float32),
            pltpu.VMEM_SHARED((N_PAD, FH), jnp.float32),
            pltpu.SemaphoreType.DMA,
        ],
    )
    def sc_degree(dst_hbm, ones_hbm, zeros_hbm, out_hbm, dst_v, ones_v, acc,
                  sem):
        c = lax.axis_index("c")
        s = lax.axis_index("s")
        w = c * NS + s
        pltpu.sync_copy(zeros_hbm, acc.at[pl.ds(s * ZR, ZR)])
        pltpu.sync_copy(ones_hbm, ones_v)
        pltpu.sync_copy(dst_hbm.at[w], dst_v)
        plsc.subcore_barrier()

        def fire(j, carry):
            pltpu.async_copy(ones_v, acc.at[dst_v.at[j]], sem, add=True)
            return carry

        lax.fori_loop(0, CH, fire, 0)

        def drain(j, carry):
            pltpu.make_async_copy(ones_v, acc.at[dst_v.at[j]], sem).wait()
            return carry

        lax.fori_loop(0, CH, drain, 0)
        plsc.subcore_barrier()
        pltpu.sync_copy(acc.at[pl.ds(s * ZR, ZR)],
                        out_hbm.at[c, pl.ds(s * ZR, ZR)])

    @functools.partial(
        pl.kernel,
        out_type=jax.ShapeDtypeStruct((NC, N_PAD, FH), jnp.float32),
        mesh=mesh,
        scratch_types=[
            pltpu.VMEM((GC, 128), jnp.int32),
            pltpu.VMEM((GC, 128), jnp.int32),
        ]
        + [pltpu.VMEM((128, FH), jnp.float32)] * NB
        + [pltpu.VMEM_SHARED((N_PAD, FH), jnp.float32)]
        + [pltpu.SemaphoreType.DMA] * (2 * NB),
    )
    def sc_mp(p_hbm, src_hbm, dst_hbm, zeros_hbm, out_hbm,
              src_v, dst_v, *rest):
        rows = rest[:NB]
        acc = rest[NB]
        gsem = rest[NB + 1:NB + 1 + NB]
        ssem = rest[NB + 1 + NB:]
        c = lax.axis_index("c")
        s = lax.axis_index("s")
        w = c * NS + s
        pltpu.sync_copy(zeros_hbm, acc.at[pl.ds(s * ZR, ZR)])
        plsc.subcore_barrier()

        for grp in range(GRP):
            pltpu.sync_copy(src_hbm.at[w, pl.ds(grp * GC, GC)], src_v)
            pltpu.sync_copy(dst_hbm.at[w, pl.ds(grp * GC, GC)], dst_v)

            for b in range(NB):
                pltpu.async_copy(p_hbm.at[src_v.at[b]], rows[b], gsem[b])

            def outer(jj, carry):
                for b in range(NB):
                    i = jj * NB + b
                    bp = (b + 1) % NB
                    npre = i + 1

                    @pl.when(jnp.logical_and(npre >= NB, npre < GC))
                    def _():
                        pltpu.make_async_copy(
                            rows[bp], acc.at[dst_v.at[0]], ssem[bp]).wait()
                        pltpu.async_copy(p_hbm.at[src_v.at[npre]], rows[bp],
                                         gsem[bp])

                    pltpu.make_async_copy(p_hbm.at[src_v.at[i]], rows[b],
                                          gsem[b]).wait()
                    pltpu.async_copy(rows[b], acc.at[dst_v.at[i]], ssem[b],
                                     add=True)
                return carry

            lax.fori_loop(0, GC // NB, outer, 0)
            for b in range(NB):
                pltpu.make_async_copy(rows[b], acc.at[dst_v.at[0]],
                                      ssem[b]).wait()
        plsc.subcore_barrier()
        pltpu.sync_copy(acc.at[pl.ds(s * ZR, ZR)],
                        out_hbm.at[c, pl.ds(s * ZR, ZR)])

    return sc_degree, sc_mp


_BLK = 1024
_NBLK = N_PAD // _BLK

_row_spec = pl.BlockSpec((_BLK, FH), lambda i: (i, 0))
_deg_spec = pl.BlockSpec((NC, _BLK, FH), lambda i: (0, i, 0))
_w_spec = pl.BlockSpec((FH, FH), lambda i: (0, 0))
_b_spec = pl.BlockSpec((1, FH), lambda i: (0, 0))
_agg_spec = pl.BlockSpec((NC, _BLK, FH), lambda i: (0, i, 0))
_a_spec = pl.BlockSpec((_BLK, 8), lambda i: (i, 0))


def _tc_prep_body(deg_ref, x_ref, w1_ref, b1_ref, w7_ref, b7_ref,
                  p1_ref, p7_ref, a_ref):
    d = deg_ref[0, :, 0:1] + deg_ref[1, :, 0:1] + 1.0
    a = lax.rsqrt(d)
    a_ref[...] = jnp.broadcast_to(a, a_ref.shape)
    xb = x_ref[...]
    h1 = jnp.dot(xb, w1_ref[...], preferred_element_type=jnp.float32) + b1_ref[...]
    h7 = jnp.dot(xb, w7_ref[...], preferred_element_type=jnp.float32) + b7_ref[...]
    p1_ref[...] = a * h1
    p7_ref[...] = a * h7


_tc_prep = pl.pallas_call(
    _tc_prep_body,
    grid=(_NBLK,),
    in_specs=[_deg_spec, _row_spec, _w_spec, _b_spec, _w_spec, _b_spec],
    out_specs=[_row_spec, _row_spec, _a_spec],
    out_shape=[jax.ShapeDtypeStruct((N_PAD, FH), jnp.float32)] * 2
    + [jax.ShapeDtypeStruct((N_PAD, 8), jnp.float32)],
)


def _tc_layer_body(a_ref, agg_ref, p_ref, w_ref, b_ref, h_ref, pn_ref):
    a = a_ref[:, 0:1]
    h = jnp.maximum(a * (agg_ref[0] + agg_ref[1] + p_ref[...]), 0.0)
    h_ref[...] = h
    pn_ref[...] = a * (jnp.dot(h, w_ref[...], preferred_element_type=jnp.float32) + b_ref[...])


_tc_layer = pl.pallas_call(
    _tc_layer_body,
    grid=(_NBLK,),
    in_specs=[_a_spec, _agg_spec, _row_spec, _w_spec, _b_spec],
    out_specs=[_row_spec, _row_spec],
    out_shape=[jax.ShapeDtypeStruct((N_PAD, FH), jnp.float32)] * 2,
)


def _tc_combine_body(a_ref, agg_ref, p_ref, h_ref):
    a = a_ref[:, 0:1]
    h_ref[...] = jnp.maximum(a * (agg_ref[0] + agg_ref[1] + p_ref[...]), 0.0)


_tc_combine = pl.pallas_call(
    _tc_combine_body,
    grid=(_NBLK,),
    in_specs=[_a_spec, _agg_spec, _row_spec],
    out_specs=_row_spec,
    out_shape=jax.ShapeDtypeStruct((N_PAD, FH), jnp.float32),
)


def _tc_pool_body(h_ref, batch_ref, out_ref):
    b = pl.program_id(0)
    hv = h_ref[...]
    mask = batch_ref[...] == b
    mx = jnp.max(jnp.where(mask, hv, -jnp.inf), axis=0, keepdims=True)
    sm = jnp.sum(jnp.where(mask, hv, 0.0), axis=0, keepdims=True)
    cnt = jnp.sum(mask.astype(jnp.float32))
    mean = sm / jnp.maximum(cnt, 1.0)
    out_ref[...] = jnp.concatenate([mx, mean], axis=1).reshape(1, 1, 2 * FH)


_tc_pool = pl.pallas_call(
    _tc_pool_body,
    grid=(B,),
    in_specs=[
        pl.BlockSpec((N_PAD, FH), lambda b: (0, 0)),
        pl.BlockSpec((N_PAD, 1), lambda b: (0, 0)),
    ],
    out_specs=pl.BlockSpec((1, 1, 2 * FH), lambda b: (b, 0, 0)),
    out_shape=jax.ShapeDtypeStruct((B, 1, 2 * FH), jnp.float32),
)


def _tc_head_body(x1, x2, x3, pa, y, l1, c1, l2, c2, l3, c3,
                  loss_ref, pred_ref, probs_ref, xaug_ref):
    r = (jnp.maximum(x1[...], 0.0) + jnp.maximum(x2[...], 0.0)
         + jnp.maximum(x3[...], 0.0))
    z = jnp.maximum(jnp.dot(r, l1[...],            preferred_element_type=jnp.float32) + c1[...], 0.0)
    z = jnp.maximum(jnp.dot(z, l2[...],            preferred_element_type=jnp.float32) + c2[...], 0.0)
    lg = jnp.dot(z, l3[...], preferred_element_type=jnp.float32) + c3[...]
    lo = lg[:, 0:1]
    hi = lg[:, 1:2]
    m = jnp.maximum(lo, hi)
    lse = m + jnp.log(jnp.exp(lo - m) + jnp.exp(hi - m))
    lp0 = lo - lse
    lp1 = hi - lse
    lpy = jnp.where(y[...] == 1, lp1, lp0)
    loss_ref[...] = jnp.reshape(-jnp.mean(lpy), (1, 1))
    pred_ref[...] = (hi > lo).astype(jnp.int32)
    probs_ref[...] = jnp.concatenate([jnp.exp(lp0), jnp.exp(lp1)], axis=1)
    pav = pa[...]
    for k in range(10):
        xaug_ref[:, 2 * FH * k:2 * FH * (k + 1)] = pav


_tc_head = pl.pallas_call(
    _tc_head_body,
    out_shape=[
        jax.ShapeDtypeStruct((1, 1), jnp.float32),
        jax.ShapeDtypeStruct((B, 1), jnp.int32),
        jax.ShapeDtypeStruct((B, C), jnp.float32),
        jax.ShapeDtypeStruct((B, 20 * FH), jnp.float32),
    ],
)


def kernel(x, edge_index, batch, y, W1, b1, W2, b2, W3, b3, W7, b7, W8, b8,
           L1, c1, L2, c2, L3, c3):
    src = edge_index[0]
    dst = edge_index[1]
    pad = (jnp.arange(E_PAD - E, dtype=jnp.int32) % (N_PAD - N)) + N
    src3 = jnp.concatenate([src, pad]).reshape(NW, CH, 128)
    dst3 = jnp.concatenate([dst, pad]).reshape(NW, CH, 128)
    x_pad = jnp.pad(x, ((0, N_PAD - N), (0, 0)))
    batch_col = jnp.pad(batch, (0, N_PAD - N), constant_values=B).reshape(N_PAD, 1)
    zeros128 = jnp.zeros((ZR, FH), jnp.float32)
    ones128 = jnp.ones((128, FH), jnp.float32)
    row = lambda v: v.reshape(1, -1)
    _sc_degree, _sc_mp = _sc_kernels()

    deg2 = _sc_degree(dst3, ones128, zeros128)
    p1, p7, a8 = _tc_prep(deg2, x_pad, W1, row(b1), W7, row(b7))

    agg1 = _sc_mp(p1, src3, dst3, zeros128)
    agg7 = _sc_mp(p7, src3, dst3, zeros128)

    h1, p2 = _tc_layer(a8, agg1, p1, W2, row(b2))
    x1 = _tc_pool(h1, batch_col)
    agg2 = _sc_mp(p2, src3, dst3, zeros128)

    h2, p3 = _tc_layer(a8, agg2, p2, W3, row(b3))
    x2 = _tc_pool(h2, batch_col)
    agg3 = _sc_mp(p3, src3, dst3, zeros128)

    h3 = _tc_combine(a8, agg3, p3)
    x3 = _tc_pool(h3, batch_col)

    ha1, p8 = _tc_layer(a8, agg7, p7, W8, row(b8))
    agg8 = _sc_mp(p8, src3, dst3, zeros128)
    ha2 = _tc_combine(a8, agg8, p8)
    pa = _tc_pool(ha2, batch_col)

    sq = lambda v: v.reshape(B, 2 * FH)
    loss, pred, probs, xaug = _tc_head(
        sq(x1), sq(x2), sq(x3), sq(pa), y.reshape(B, 1),
        L1, row(c1), L2, row(c2), L3, row(c3))
    return (loss.reshape(()), pred.reshape(B), probs, xaug)
